# baseline (speedup 1.0000x reference)
import functools
import jax
import jax.numpy as jnp
from jax import lax
from jax.experimental import pallas as pl
from jax.experimental.pallas import tpu as pltpu
from jax.experimental.pallas import tpu_sc as plsc

_NC = 2
_NS = 16
_NW = _NC * _NS
_CHUNK = 32
_LANES = 16


def _sc_body(batch, num_types, dim, emb_hbm, out_hbm, buf, sem):
    c = lax.axis_index("c")
    s = lax.axis_index("s")
    wid = s * _NC + c
    rows_per_tile = batch // _NW
    pltpu.async_copy(emb_hbm, buf.at[:, 0, :], sem).wait()
    nvec = dim // _LANES
    regs = [
        buf[t, 0, pl.ds(l * _LANES, _LANES)]
        for t in range(num_types)
        for l in range(nvec)
    ]

    def _rep(r, carry):
        for t in range(num_types):
            for l in range(nvec):
                buf[t, r, pl.ds(l * _LANES, _LANES)] = regs[t * nvec + l]
        return carry

    lax.fori_loop(1, _CHUNK, _rep, 0)

    base = wid * rows_per_tile
    outs = [
        pltpu.async_copy(
            buf, out_hbm.at[:, pl.ds(base + i * _CHUNK, _CHUNK), :], sem
        )
        for i in range(rows_per_tile // _CHUNK)
    ]
    for cp in outs:
        cp.wait()


def kernel(action_mask, embedding):
    batch = action_mask.shape[0]
    num_types, dim = embedding.shape
    mesh = plsc.VectorSubcoreMesh(core_axis_name="c", subcore_axis_name="s")
    body = functools.partial(_sc_body, batch, num_types, dim)
    run = pl.kernel(
        body,
        out_type=jax.ShapeDtypeStruct((num_types, batch, dim), embedding.dtype),
        mesh=mesh,
        scratch_types=[
            pltpu.VMEM((num_types, _CHUNK, dim), embedding.dtype),
            pltpu.SemaphoreType.DMA,
        ],
        compiler_params=pltpu.CompilerParams(use_tc_tiling_on_sc=True),
    )
    return run(embedding).transpose(1, 0, 2)


# speedup vs baseline: 1.0257x; 1.0257x over previous
import functools
import jax
import jax.numpy as jnp
from jax import lax
from jax.experimental import pallas as pl
from jax.experimental.pallas import tpu as pltpu
from jax.experimental.pallas import tpu_sc as plsc

_NC = 2
_NS = 16
_NW = _NC * _NS
_CHUNK = 16
_LANES = 16


def _sc_body(batch, num_types, dim, emb_hbm, out_hbm, buf, sem):
    c = lax.axis_index("c")
    s = lax.axis_index("s")
    wid = s * _NC + c
    rows_per_tile = batch // _NW
    pltpu.async_copy(emb_hbm, buf.at[:, 0, :], sem).wait()
    nvec = dim // _LANES
    regs = [
        buf[t, 0, pl.ds(l * _LANES, _LANES)]
        for t in range(num_types)
        for l in range(nvec)
    ]

    def _rep(r, carry):
        for t in range(num_types):
            for l in range(nvec):
                buf[t, r, pl.ds(l * _LANES, _LANES)] = regs[t * nvec + l]
        return carry

    lax.fori_loop(1, _CHUNK, _rep, 0)

    base = wid * rows_per_tile
    outs = [
        pltpu.async_copy(
            buf, out_hbm.at[:, pl.ds(base + i * _CHUNK, _CHUNK), :], sem
        )
        for i in range(rows_per_tile // _CHUNK)
    ]
    for cp in outs:
        cp.wait()


def kernel(action_mask, embedding):
    batch = action_mask.shape[0]
    num_types, dim = embedding.shape
    mesh = plsc.VectorSubcoreMesh(core_axis_name="c", subcore_axis_name="s")
    body = functools.partial(_sc_body, batch, num_types, dim)
    run = pl.kernel(
        body,
        out_type=jax.ShapeDtypeStruct((num_types, batch, dim), embedding.dtype),
        mesh=mesh,
        scratch_types=[
            pltpu.VMEM((num_types, _CHUNK, dim), embedding.dtype),
            pltpu.SemaphoreType.DMA,
        ],
        compiler_params=pltpu.CompilerParams(use_tc_tiling_on_sc=True),
    )
    return run(embedding).transpose(1, 0, 2)
